# baseline (speedup 1.0000x reference)
import functools

import jax
import jax.numpy as jnp
from jax import lax
from jax.experimental import pallas as pl
from jax.experimental.pallas import tpu as pltpu
from jax.experimental.pallas import tpu_sc as plsc

L0, L1, L2, L3 = 16, 24, 24, 16
OFF_X, OFF_B1, OFF_B2, OFF_B3 = 0, 16, 48, 80
OFF_W1, OFF_W2, OFF_W3 = 96, 608, 1376
PACKED = 1760


def _bcast(v, i):
    idx = jnp.full((16,), i, dtype=jnp.int32)
    return v.at[idx].get(mode="promise_in_bounds", indices_are_sorted=True)


def _silu(z):
    return z / (1.0 + jnp.exp(-z))


def _body(p_h, out_h, p_v, out_v):
    c = lax.axis_index("c")
    s = lax.axis_index("s")

    @pl.when(jnp.logical_and(c == 0, s == 0))
    def _():
        pltpu.sync_copy(p_h, p_v)

        x = p_v[pl.ds(OFF_X, 16)]
        h1a = p_v[pl.ds(OFF_B1, 16)]
        h1b = p_v[pl.ds(OFF_B1 + 16, 16)]
        for i in range(L0):
            sc = _bcast(x, i)
            h1a = h1a + sc * p_v[pl.ds(OFF_W1 + 32 * i, 16)]
            h1b = h1b + sc * p_v[pl.ds(OFF_W1 + 32 * i + 16, 16)]
        h1a = _silu(h1a)
        h1b = _silu(h1b)

        h2a = p_v[pl.ds(OFF_B2, 16)]
        h2b = p_v[pl.ds(OFF_B2 + 16, 16)]
        for i in range(L1):
            sc = _bcast(h1a, i) if i < 16 else _bcast(h1b, i - 16)
            h2a = h2a + sc * p_v[pl.ds(OFF_W2 + 32 * i, 16)]
            h2b = h2b + sc * p_v[pl.ds(OFF_W2 + 32 * i + 16, 16)]
        h2a = _silu(h2a)
        h2b = _silu(h2b)

        y = p_v[pl.ds(OFF_B3, 16)]
        for i in range(L2):
            sc = _bcast(h2a, i) if i < 16 else _bcast(h2b, i - 16)
            y = y + sc * p_v[pl.ds(OFF_W3 + 16 * i, 16)]

        out_v[...] = y
        pltpu.sync_copy(out_v, out_h)


@functools.partial(
    pl.kernel,
    out_type=jax.ShapeDtypeStruct((L3,), jnp.float32),
    mesh=plsc.VectorSubcoreMesh(core_axis_name="c", subcore_axis_name="s"),
    scratch_types=[
        pltpu.VMEM((PACKED,), jnp.float32),
        pltpu.VMEM((L3,), jnp.float32),
    ],
)
def _mlp_sc(p_h, out_h, p_v, out_v):
    _body(p_h, out_h, p_v, out_v)


def kernel(x, W1, W2, W3, bias):
    f32 = jnp.float32
    w1t = jnp.zeros((L0, 32), f32).at[:, :L1].set(W1.T)
    w2t = jnp.zeros((L1, 32), f32).at[:, :L2].set(W2.T)
    w3t = W3.T
    b1 = jnp.zeros((32,), f32).at[:L1].set(bias[16:40])
    b2 = jnp.zeros((32,), f32).at[:L2].set(bias[40:64])
    packed = jnp.concatenate([
        x, b1, b2, bias[64:80],
        w1t.reshape(-1), w2t.reshape(-1), w3t.reshape(-1),
    ])
    return _mlp_sc(packed)


# speedup vs baseline: 7.0165x; 7.0165x over previous
import jax
import jax.numpy as jnp
from jax.experimental import pallas as pl


def _silu(z):
    return z / (1.0 + jnp.exp(-z))


def _mlp_body(x_ref, w1_ref, w2_ref, w3_ref, b_ref, out_ref):
    x = x_ref[...]
    b = b_ref[...]
    h1 = _silu(jnp.sum(w1_ref[...] * x[None, :], axis=1) + b[16:40])
    h2 = _silu(jnp.sum(w2_ref[...] * h1[None, :], axis=1) + b[40:64])
    y = jnp.sum(w3_ref[...] * h2[None, :], axis=1) + b[64:80]
    out_ref[...] = y


def kernel(x, W1, W2, W3, bias):
    return pl.pallas_call(
        _mlp_body,
        out_shape=jax.ShapeDtypeStruct((16,), jnp.float32),
    )(x, W1, W2, W3, bias)


# speedup vs baseline: 7.1381x; 1.0173x over previous
import jax
import jax.numpy as jnp
from jax.experimental import pallas as pl
from jax.experimental.pallas import tpu as pltpu


def _silu(z):
    return z / (1.0 + jnp.exp(-z))


def _mlp_body(x_h, w1_h, w2_h, w3_h, b_h, out_ref,
              x_v, w1_v, w2_v, w3_v, b_v, sems):
    cp_x = pltpu.make_async_copy(x_h, x_v, sems.at[0])
    cp_b = pltpu.make_async_copy(b_h, b_v, sems.at[1])
    cp_w1 = pltpu.make_async_copy(w1_h, w1_v, sems.at[2])
    cp_w2 = pltpu.make_async_copy(w2_h, w2_v, sems.at[3])
    cp_w3 = pltpu.make_async_copy(w3_h, w3_v, sems.at[4])
    for cp in (cp_x, cp_b, cp_w1, cp_w2, cp_w3):
        cp.start()

    cp_x.wait()
    cp_b.wait()
    cp_w1.wait()
    x = x_v[...]
    b = b_v[...]
    h1 = _silu(jnp.sum(w1_v[...] * x[None, :], axis=1) + b[16:40])
    cp_w2.wait()
    h2 = _silu(jnp.sum(w2_v[...] * h1[None, :], axis=1) + b[40:64])
    cp_w3.wait()
    out_ref[...] = jnp.sum(w3_v[...] * h2[None, :], axis=1) + b[64:80]


def kernel(x, W1, W2, W3, bias):
    return pl.pallas_call(
        _mlp_body,
        in_specs=[pl.BlockSpec(memory_space=pl.ANY)] * 5,
        out_shape=jax.ShapeDtypeStruct((16,), jnp.float32),
        scratch_shapes=[
            pltpu.VMEM((16,), jnp.float32),
            pltpu.VMEM((24, 16), jnp.float32),
            pltpu.VMEM((24, 24), jnp.float32),
            pltpu.VMEM((16, 24), jnp.float32),
            pltpu.VMEM((80,), jnp.float32),
            pltpu.SemaphoreType.DMA((5,)),
        ],
    )(x, W1, W2, W3, bias)
